# Initial kernel scaffold; baseline (speedup 1.0000x reference)
import jax
import jax.numpy as jnp
from jax.experimental import pallas as pl


def kernel(x, y, hyperedge_index, W1, W2):
    raise NotImplementedError("write your pallas kernel here")


# speedup vs baseline: 5.7909x; 5.7909x over previous
import functools

import jax
import jax.numpy as jnp
from jax import lax
from jax.experimental import pallas as pl
from jax.experimental.pallas import tpu as pltpu
from jax.experimental.pallas import tpu_sc as plsc

NC, NS = 2, 16
NW = NC * NS
K = 128
D = 128
BLK = 2048


def _round_up(a, b):
    return (a + b - 1) // b * b


def _mm_a(x_ref, w_ref, o_ref):
    o_ref[...] = jnp.dot(x_ref[...], w_ref[...],
                         preferred_element_type=jnp.float32)


def _stage_b(e0, e1, d0, d1, yb, xwb, xb, w2, etop_o, ew_o, ewself_o):
    deg = jnp.maximum(d0[...] + d1[...], 1.0)
    eagg = (e0[...] + e1[...]) / deg
    etop = jnp.maximum(eagg + yb[...], 0.0)
    etop_o[...] = etop
    ew_o[...] = jnp.dot(etop, w2[...], preferred_element_type=jnp.float32)
    eself = jnp.maximum(xwb[...] + xb[...], 0.0)
    ewself_o[...] = jnp.dot(eself, w2[...], preferred_element_type=jnp.float32)


def _stage_c(n0, n1, ews, v0, v1, n_o):
    deg = v0[...] + v1[...] + 1.0
    acc = n0[...] + n1[...] + ews[...]
    n_o[...] = jnp.maximum(acc / deg, 0.0)


def _sc_pass1(np_, chunks):
    z = np_ // NS

    def body(xw_hbm, src_hbm, dst_hbm, z2d, z1d,
             eacc_out, dege_out, degv_out,
             idx_s, idx_d, rows, ones_v, eacc_sh, dege_sh, degv_sh, sem):
        c = lax.axis_index("c")
        s = lax.axis_index("s")
        zoff = s * z
        pltpu.sync_copy(z2d.at[pl.ds(zoff, z)], eacc_sh.at[pl.ds(zoff, z)])
        pltpu.sync_copy(z1d.at[pl.ds(zoff, z)], dege_sh.at[pl.ds(zoff, z)])
        pltpu.sync_copy(z1d.at[pl.ds(zoff, z)], degv_sh.at[pl.ds(zoff, z)])
        for i in range(K // 16):
            ones_v[pl.ds(i * 16, 16)] = jnp.ones((16,), jnp.float32)
        plsc.subcore_barrier()
        wid = s * NC + c
        base = wid * (chunks * K)

        def step(j, carry):
            off = pl.multiple_of(base + j * K, 8)
            pltpu.sync_copy(src_hbm.at[pl.ds(off, K)], idx_s)
            pltpu.sync_copy(dst_hbm.at[pl.ds(off, K)], idx_d)
            pltpu.async_copy(xw_hbm.at[idx_s], rows, sem).wait()
            pltpu.sync_copy(rows, eacc_sh.at[idx_d], add=True)
            pltpu.sync_copy(ones_v, dege_sh.at[idx_d], add=True)
            pltpu.sync_copy(ones_v, degv_sh.at[idx_s], add=True)
            return carry

        lax.fori_loop(0, chunks, step, 0)
        plsc.subcore_barrier()
        pltpu.sync_copy(eacc_sh.at[pl.ds(zoff, z)],
                        eacc_out.at[c, pl.ds(zoff, z)])
        pltpu.sync_copy(dege_sh.at[pl.ds(zoff, z)],
                        dege_out.at[c, pl.ds(zoff, z)])
        pltpu.sync_copy(degv_sh.at[pl.ds(zoff, z)],
                        degv_out.at[c, pl.ds(zoff, z)])

    return pl.kernel(
        body,
        out_type=(
            jax.ShapeDtypeStruct((NC, np_, D), jnp.float32),
            jax.ShapeDtypeStruct((NC, np_), jnp.float32),
            jax.ShapeDtypeStruct((NC, np_), jnp.float32),
        ),
        mesh=plsc.VectorSubcoreMesh(core_axis_name="c", subcore_axis_name="s"),
        scratch_types=[
            pltpu.VMEM((K,), jnp.int32),
            pltpu.VMEM((K,), jnp.int32),
            pltpu.VMEM((K, D), jnp.float32),
            pltpu.VMEM((K,), jnp.float32),
            pltpu.VMEM_SHARED((np_, D), jnp.float32),
            pltpu.VMEM_SHARED((np_,), jnp.float32),
            pltpu.VMEM_SHARED((np_,), jnp.float32),
            pltpu.SemaphoreType.DMA,
        ],
    )


def _sc_pass2(np_, chunks):
    z = np_ // NS

    def body(ew_hbm, src_hbm, dst_hbm, z2d,
             nacc_out,
             idx_s, idx_d, rows, nacc_sh, sem):
        c = lax.axis_index("c")
        s = lax.axis_index("s")
        zoff = s * z
        pltpu.sync_copy(z2d.at[pl.ds(zoff, z)], nacc_sh.at[pl.ds(zoff, z)])
        plsc.subcore_barrier()
        wid = s * NC + c
        base = wid * (chunks * K)

        def step(j, carry):
            off = pl.multiple_of(base + j * K, 8)
            pltpu.sync_copy(src_hbm.at[pl.ds(off, K)], idx_s)
            pltpu.sync_copy(dst_hbm.at[pl.ds(off, K)], idx_d)
            pltpu.async_copy(ew_hbm.at[idx_d], rows, sem).wait()
            pltpu.sync_copy(rows, nacc_sh.at[idx_s], add=True)
            return carry

        lax.fori_loop(0, chunks, step, 0)
        plsc.subcore_barrier()
        pltpu.sync_copy(nacc_sh.at[pl.ds(zoff, z)],
                        nacc_out.at[c, pl.ds(zoff, z)])

    return pl.kernel(
        body,
        out_type=jax.ShapeDtypeStruct((NC, np_, D), jnp.float32),
        mesh=plsc.VectorSubcoreMesh(core_axis_name="c", subcore_axis_name="s"),
        scratch_types=[
            pltpu.VMEM((K,), jnp.int32),
            pltpu.VMEM((K,), jnp.int32),
            pltpu.VMEM((K, D), jnp.float32),
            pltpu.VMEM_SHARED((np_, D), jnp.float32),
            pltpu.SemaphoreType.DMA,
        ],
    )


def kernel(x, y, hyperedge_index, W1, W2):
    num_nodes = x.shape[0]
    num_edges = y.shape[0]
    nnz = hyperedge_index.shape[1]

    np_ = _round_up(max(num_nodes, num_edges), BLK)
    nnzp = _round_up(nnz, K * NW)
    chunks = nnzp // (K * NW)
    pad_slot = max(num_nodes, num_edges) + 16

    x_p = jnp.zeros((np_, D), jnp.float32).at[:num_nodes].set(x)
    y_p = jnp.zeros((np_, D), jnp.float32).at[:num_edges].set(y)
    padv = jnp.full((nnzp - nnz,), pad_slot, jnp.int32)
    src = jnp.concatenate([hyperedge_index[0].astype(jnp.int32), padv])
    dst = jnp.concatenate([hyperedge_index[1].astype(jnp.int32), padv])
    z2d = jnp.zeros((np_, D), jnp.float32)
    z1d = jnp.zeros((np_,), jnp.float32)

    grid = np_ // BLK
    row_spec = pl.BlockSpec((BLK, D), lambda i: (i, 0))
    col_spec = pl.BlockSpec((BLK, 1), lambda i: (i, 0))
    w_spec = pl.BlockSpec((D, D), lambda i: (0, 0))

    xw_p = pl.pallas_call(
        _mm_a,
        grid=(grid,),
        in_specs=[row_spec, w_spec],
        out_specs=row_spec,
        out_shape=jax.ShapeDtypeStruct((np_, D), jnp.float32),
    )(x_p, W1)

    eacc, dege, degv = _sc_pass1(np_, chunks)(xw_p, src, dst, z2d, z1d)

    etop, ew, ewself = pl.pallas_call(
        _stage_b,
        grid=(grid,),
        in_specs=[row_spec, row_spec, col_spec, col_spec,
                  row_spec, row_spec, row_spec, w_spec],
        out_specs=[row_spec, row_spec, row_spec],
        out_shape=[jax.ShapeDtypeStruct((np_, D), jnp.float32)] * 3,
    )(eacc[0], eacc[1], dege[0].reshape(np_, 1), dege[1].reshape(np_, 1),
      y_p, xw_p, x_p, W2)

    nacc = _sc_pass2(np_, chunks)(ew, src, dst, z2d)

    n = pl.pallas_call(
        _stage_c,
        grid=(grid,),
        in_specs=[row_spec, row_spec, row_spec, col_spec, col_spec],
        out_specs=row_spec,
        out_shape=jax.ShapeDtypeStruct((np_, D), jnp.float32),
    )(nacc[0], nacc[1], ewself,
      degv[0].reshape(np_, 1), degv[1].reshape(np_, 1))

    return (n[:num_nodes], etop[:num_edges])
